# Initial kernel scaffold; baseline (speedup 1.0000x reference)
import jax
import jax.numpy as jnp
from jax.experimental import pallas as pl


def kernel(x, edge_index, W1, b1, W2, b2):
    raise NotImplementedError("write your pallas kernel here")


# speedup vs baseline: 19.6292x; 19.6292x over previous
import functools

import jax
import jax.numpy as jnp
from jax import lax
from jax.experimental import pallas as pl
from jax.experimental.pallas import tpu as pltpu
from jax.experimental.pallas import tpu_sc as plsc

_NC = 2
_NS = 16
_W = _NC * _NS
_K = 80
_LANES = 16
_NPAD = 10240


def _zero_fill(ref, nrows, ncols):
  z = jnp.zeros((_LANES,), jnp.float32)

  def body(r, carry):
    for c in range(ncols // _LANES):
      ref[r, pl.ds(c * _LANES, _LANES)] = z
    return carry

  lax.fori_loop(0, nrows, body, 0)


def _ones_fill(ref, nrows, ncols):
  o = jnp.ones((_LANES,), jnp.float32)

  def body(r, carry):
    for c in range(ncols // _LANES):
      ref[r, pl.ds(c * _LANES, _LANES)] = o
    return carry

  lax.fori_loop(0, nrows, body, 0)


def _make_deg_kernel(e):
  n = _NPAD
  epw = e // _W
  c_chunks = epw // _K
  rpt = n // _NS
  zrows = 16
  mesh = plsc.VectorSubcoreMesh(core_axis_name="c", subcore_axis_name="s")

  def body(dst_hbm, out_hbm, acc_sp, zb, onesb, dstv):
    cid = lax.axis_index("c")
    sid = lax.axis_index("s")
    w = cid * _NS + sid
    base = sid * rpt
    _zero_fill(zb, zrows, _LANES)

    def zcopy(i, carry):
      off = pl.multiple_of(base + i * zrows, zrows)
      pltpu.sync_copy(zb, acc_sp.at[pl.ds(off, zrows), :])
      return carry

    lax.fori_loop(0, rpt // zrows, zcopy, 0)
    _ones_fill(onesb, _K, _LANES)
    pltpu.sync_copy(dst_hbm.at[w], dstv)
    plsc.subcore_barrier()

    def chunk(j, carry):
      pltpu.sync_copy(onesb, acc_sp.at[dstv.at[j]], add=True)
      return carry

    lax.fori_loop(0, c_chunks, chunk, 0)
    plsc.subcore_barrier()
    pltpu.sync_copy(acc_sp.at[pl.ds(base, rpt), :],
                    out_hbm.at[cid, pl.ds(base, rpt), :])

  return pl.kernel(
      body,
      out_type=jax.ShapeDtypeStruct((_NC, n, _LANES), jnp.float32),
      mesh=mesh,
      scratch_types=[
          pltpu.VMEM_SHARED((n, _LANES), jnp.float32),
          pltpu.VMEM((zrows, _LANES), jnp.float32),
          pltpu.VMEM((_K, _LANES), jnp.float32),
          pltpu.VMEM((c_chunks, _K), jnp.int32),
      ],
  )


def _make_agg_kernel(e, d):
  n = _NPAD
  epw = e // _W
  c_chunks = epw // _K
  rpt = n // _NS
  zrows = 16
  mesh = plsc.VectorSubcoreMesh(core_axis_name="c", subcore_axis_name="s")

  def body(src_hbm, dst_hbm, h_hbm, out_hbm, acc_sp, zb, dstv, srcv, rows,
           sem):
    cid = lax.axis_index("c")
    sid = lax.axis_index("s")
    w = cid * _NS + sid
    base = sid * rpt
    _zero_fill(zb, zrows, d)

    def zcopy(i, carry):
      off = pl.multiple_of(base + i * zrows, zrows)
      pltpu.sync_copy(zb, acc_sp.at[pl.ds(off, zrows), :])
      return carry

    lax.fori_loop(0, rpt // zrows, zcopy, 0)
    pltpu.sync_copy(src_hbm.at[pl.ds(w * epw, epw)], srcv)
    pltpu.sync_copy(dst_hbm.at[w], dstv)
    plsc.subcore_barrier()

    def chunk(j, carry):
      off = pl.multiple_of(j * _K, _K)
      pltpu.async_copy(h_hbm.at[srcv.at[pl.ds(off, _K)]], rows, sem).wait()
      pltpu.sync_copy(rows, acc_sp.at[dstv.at[j]], add=True)
      return carry

    lax.fori_loop(0, c_chunks, chunk, 0)
    plsc.subcore_barrier()
    pltpu.sync_copy(acc_sp.at[pl.ds(base, rpt), :],
                    out_hbm.at[cid, pl.ds(base, rpt), :])

  return pl.kernel(
      body,
      out_type=jax.ShapeDtypeStruct((_NC, n, d), jnp.float32),
      mesh=mesh,
      scratch_types=[
          pltpu.VMEM_SHARED((n, d), jnp.float32),
          pltpu.VMEM((zrows, d), jnp.float32),
          pltpu.VMEM((c_chunks, _K), jnp.int32),
          pltpu.VMEM((epw,), jnp.int32),
          pltpu.VMEM((_K, d), jnp.float32),
          pltpu.SemaphoreType.DMA,
      ],
  )


_BR = 1024


def _dinv_from(degp):
  deg = jnp.sum(degp, axis=(0, 2)) + 1.0
  return lax.rsqrt(deg)


def _tc_b_body(x_ref, w_ref, degp_ref, out_ref):
  dinv = _dinv_from(degp_ref[...])
  h = jnp.dot(x_ref[...], w_ref[...], preferred_element_type=jnp.float32)
  out_ref[...] = h * dinv[:, None]


def _tc_d_body(accp_ref, hs1_ref, degp_ref, b1_ref, w2_ref, out_ref):
  dinv = _dinv_from(degp_ref[...])
  accp = accp_ref[...]
  z = dinv[:, None] * (accp[0] + accp[1] + hs1_ref[...]) + b1_ref[...]
  s = jax.nn.sigmoid(z)
  out_ref[...] = jnp.dot(
      s, w2_ref[...], preferred_element_type=jnp.float32) * dinv[:, None]


def _tc_f_body(accp_ref, hs2_ref, degp_ref, b2_ref, out_ref, *, d_out):
  dinv = _dinv_from(degp_ref[...])
  accp = accp_ref[...]
  z = dinv[:, None] * (accp[0] + accp[1] + hs2_ref[...]) + b2_ref[...]
  col = lax.broadcasted_iota(jnp.int32, z.shape, 1)
  mask = col < d_out
  zm = jnp.where(mask, z, jnp.float32(-1e30))
  m = jnp.max(zm, axis=1, keepdims=True)
  ex = jnp.where(mask, jnp.exp(zm - m), 0.0)
  lse = jnp.log(jnp.sum(ex, axis=1, keepdims=True))
  out_ref[...] = zm - m - lse


def kernel(x, edge_index, W1, b1, W2, b2):
  n, d_in = x.shape
  e = edge_index.shape[1]
  d_hid = W1.shape[1]
  d_out = W2.shape[1]
  d2 = 128
  npad = _NPAD

  xp = jnp.pad(x, ((0, npad - n), (0, 0)))
  src = edge_index[0]
  dst3d = edge_index[1].reshape(_W, -1, _K)

  w2p = jnp.pad(W2, ((0, 0), (0, d2 - d_out)))
  b1r = b1.reshape(1, d_hid)
  b2r = jnp.pad(b2, (0, d2 - d_out)).reshape(1, d2)

  degp = _make_deg_kernel(e)(dst3d)

  grid = (npad // _BR,)
  hs1 = pl.pallas_call(
      _tc_b_body,
      grid=grid,
      in_specs=[
          pl.BlockSpec((_BR, d_in), lambda i: (i, 0)),
          pl.BlockSpec((d_in, d_hid), lambda i: (0, 0)),
          pl.BlockSpec((_NC, _BR, _LANES), lambda i: (0, i, 0)),
      ],
      out_specs=pl.BlockSpec((_BR, d_hid), lambda i: (i, 0)),
      out_shape=jax.ShapeDtypeStruct((npad, d_hid), jnp.float32),
  )(xp, W1, degp)

  acc1 = _make_agg_kernel(e, d_hid)(src, dst3d, hs1)

  hs2 = pl.pallas_call(
      _tc_d_body,
      grid=grid,
      in_specs=[
          pl.BlockSpec((_NC, _BR, d_hid), lambda i: (0, i, 0)),
          pl.BlockSpec((_BR, d_hid), lambda i: (i, 0)),
          pl.BlockSpec((_NC, _BR, _LANES), lambda i: (0, i, 0)),
          pl.BlockSpec((1, d_hid), lambda i: (0, 0)),
          pl.BlockSpec((d_hid, d2), lambda i: (0, 0)),
      ],
      out_specs=pl.BlockSpec((_BR, d2), lambda i: (i, 0)),
      out_shape=jax.ShapeDtypeStruct((npad, d2), jnp.float32),
  )(acc1, hs1, degp, b1r, w2p)

  acc2 = _make_agg_kernel(e, d2)(src, dst3d, hs2)

  out64 = pl.pallas_call(
      functools.partial(_tc_f_body, d_out=d_out),
      grid=grid,
      in_specs=[
          pl.BlockSpec((_NC, _BR, d2), lambda i: (0, i, 0)),
          pl.BlockSpec((_BR, d2), lambda i: (i, 0)),
          pl.BlockSpec((_NC, _BR, _LANES), lambda i: (0, i, 0)),
          pl.BlockSpec((1, d2), lambda i: (0, 0)),
      ],
      out_specs=pl.BlockSpec((_BR, d2), lambda i: (i, 0)),
      out_shape=jax.ShapeDtypeStruct((npad, d2), jnp.float32),
  )(acc2, hs2, degp, b2r)

  return out64[:n, :d_out]


# speedup vs baseline: 23.8286x; 1.2139x over previous
import functools

import jax
import jax.numpy as jnp
from jax import lax
from jax.experimental import pallas as pl
from jax.experimental.pallas import tpu as pltpu
from jax.experimental.pallas import tpu_sc as plsc

_NC = 2
_NS = 16
_W = _NC * _NS
_K = 80
_LANES = 16
_NPAD = 10240


def _zero_fill(ref, nrows, ncols):
  z = jnp.zeros((_LANES,), jnp.float32)

  def body(r, carry):
    for c in range(ncols // _LANES):
      ref[r, pl.ds(c * _LANES, _LANES)] = z
    return carry

  lax.fori_loop(0, nrows, body, 0)


def _ones_fill(ref, nrows, ncols):
  o = jnp.ones((_LANES,), jnp.float32)

  def body(r, carry):
    for c in range(ncols // _LANES):
      ref[r, pl.ds(c * _LANES, _LANES)] = o
    return carry

  lax.fori_loop(0, nrows, body, 0)


def _make_deg_kernel(e):
  n = _NPAD
  epw = e // _W
  c_chunks = epw // _K
  rpt = n // _NS
  zrows = 16
  mesh = plsc.VectorSubcoreMesh(core_axis_name="c", subcore_axis_name="s")

  def body(dst_hbm, out_hbm, acc_sp, zb, onesb, dstv):
    cid = lax.axis_index("c")
    sid = lax.axis_index("s")
    w = cid * _NS + sid
    base = sid * rpt
    _zero_fill(zb, zrows, _LANES)

    def zcopy(i, carry):
      off = pl.multiple_of(base + i * zrows, zrows)
      pltpu.sync_copy(zb, acc_sp.at[pl.ds(off, zrows), :])
      return carry

    lax.fori_loop(0, rpt // zrows, zcopy, 0)
    _ones_fill(onesb, _K, _LANES)
    pltpu.sync_copy(dst_hbm.at[w], dstv)
    plsc.subcore_barrier()

    def chunk(j, carry):
      pltpu.sync_copy(onesb, acc_sp.at[dstv.at[j]], add=True)
      return carry

    lax.fori_loop(0, c_chunks, chunk, 0)
    plsc.subcore_barrier()
    pltpu.sync_copy(acc_sp.at[pl.ds(base, rpt), :],
                    out_hbm.at[cid, pl.ds(base, rpt), :])

  return pl.kernel(
      body,
      out_type=jax.ShapeDtypeStruct((_NC, n, _LANES), jnp.float32),
      mesh=mesh,
      scratch_types=[
          pltpu.VMEM_SHARED((n, _LANES), jnp.float32),
          pltpu.VMEM((zrows, _LANES), jnp.float32),
          pltpu.VMEM((_K, _LANES), jnp.float32),
          pltpu.VMEM((c_chunks, _K), jnp.int32),
      ],
  )


def _make_agg_kernel(e, d):
  n = _NPAD
  epw = e // _W
  c_chunks = epw // _K
  assert c_chunks % 2 == 1
  rpt = n // _NS
  zrows = 16
  mesh = plsc.VectorSubcoreMesh(core_axis_name="c", subcore_axis_name="s")

  def body(src_hbm, dst_hbm, h_hbm, out_hbm, acc_sp, zb, dstv, srcv, rows_a,
           rows_b, sem_a, sem_b, sem_z):
    cid = lax.axis_index("c")
    sid = lax.axis_index("s")
    w = cid * _NS + sid
    base = sid * rpt
    _zero_fill(zb, zrows, d)

    def zcopy(i, carry):
      off = pl.multiple_of(base + i * zrows, zrows)
      pltpu.sync_copy(zb, acc_sp.at[pl.ds(off, zrows), :])
      return carry

    lax.fori_loop(0, rpt // zrows, zcopy, 0)
    pltpu.sync_copy(src_hbm.at[pl.ds(w * epw, epw)], srcv)
    pltpu.sync_copy(dst_hbm.at[w], dstv)
    plsc.subcore_barrier()

    def gather(j, rows, sem):
      off = pl.multiple_of(j * _K, _K)
      return pltpu.make_async_copy(
          h_hbm.at[srcv.at[pl.ds(off, _K)]], rows, sem)

    def scatter(j, rows):
      pltpu.sync_copy(rows, acc_sp.at[dstv.at[j]], add=True)

    def pair(t, carry):
      ja = 2 * t
      jb = 2 * t + 1
      da = gather(ja, rows_a, sem_a)
      db = gather(jb, rows_b, sem_b)
      da.start()
      db.start()
      da.wait()
      scatter(ja, rows_a)
      db.wait()
      scatter(jb, rows_b)
      return carry

    lax.fori_loop(0, (c_chunks - 1) // 2, pair, 0)
    dl = gather(c_chunks - 1, rows_a, sem_a)
    dl.start()
    dl.wait()
    scatter(c_chunks - 1, rows_a)
    plsc.subcore_barrier()
    pltpu.sync_copy(acc_sp.at[pl.ds(base, rpt), :],
                    out_hbm.at[cid, pl.ds(base, rpt), :])

  return pl.kernel(
      body,
      out_type=jax.ShapeDtypeStruct((_NC, n, d), jnp.float32),
      mesh=mesh,
      scratch_types=[
          pltpu.VMEM_SHARED((n, d), jnp.float32),
          pltpu.VMEM((zrows, d), jnp.float32),
          pltpu.VMEM((c_chunks, _K), jnp.int32),
          pltpu.VMEM((epw,), jnp.int32),
          pltpu.VMEM((_K, d), jnp.float32),
          pltpu.VMEM((_K, d), jnp.float32),
          pltpu.SemaphoreType.DMA,
          pltpu.SemaphoreType.DMA,
          pltpu.SemaphoreType.DMA,
      ],
  )


_BR = 1024


def _dinv_from(degp):
  deg = jnp.sum(degp, axis=(0, 2)) + 1.0
  return lax.rsqrt(deg)


def _tc_b_body(x_ref, w_ref, degp_ref, out_ref):
  dinv = _dinv_from(degp_ref[...])
  h = jnp.dot(x_ref[...], w_ref[...], preferred_element_type=jnp.float32)
  out_ref[...] = h * dinv[:, None]


def _tc_d_body(accp_ref, hs1_ref, degp_ref, b1_ref, w2_ref, out_ref):
  dinv = _dinv_from(degp_ref[...])
  accp = accp_ref[...]
  z = dinv[:, None] * (accp[0] + accp[1] + hs1_ref[...]) + b1_ref[...]
  s = jax.nn.sigmoid(z)
  out_ref[...] = jnp.dot(
      s, w2_ref[...], preferred_element_type=jnp.float32) * dinv[:, None]


def _tc_f_body(accp_ref, hs2_ref, degp_ref, b2_ref, out_ref, *, d_out):
  dinv = _dinv_from(degp_ref[...])
  accp = accp_ref[...]
  z = dinv[:, None] * (accp[0] + accp[1] + hs2_ref[...]) + b2_ref[...]
  col = lax.broadcasted_iota(jnp.int32, z.shape, 1)
  mask = col < d_out
  zm = jnp.where(mask, z, jnp.float32(-1e30))
  m = jnp.max(zm, axis=1, keepdims=True)
  ex = jnp.where(mask, jnp.exp(zm - m), 0.0)
  lse = jnp.log(jnp.sum(ex, axis=1, keepdims=True))
  out_ref[...] = zm - m - lse


def kernel(x, edge_index, W1, b1, W2, b2):
  n, d_in = x.shape
  e = edge_index.shape[1]
  d_hid = W1.shape[1]
  d_out = W2.shape[1]
  d2 = 128
  npad = _NPAD

  xp = jnp.pad(x, ((0, npad - n), (0, 0)))
  src = edge_index[0]
  dst3d = edge_index[1].reshape(_W, -1, _K)

  w2p = jnp.pad(W2, ((0, 0), (0, d2 - d_out)))
  b1r = b1.reshape(1, d_hid)
  b2r = jnp.pad(b2, (0, d2 - d_out)).reshape(1, d2)

  degp = _make_deg_kernel(e)(dst3d)

  grid = (npad // _BR,)
  hs1 = pl.pallas_call(
      _tc_b_body,
      grid=grid,
      in_specs=[
          pl.BlockSpec((_BR, d_in), lambda i: (i, 0)),
          pl.BlockSpec((d_in, d_hid), lambda i: (0, 0)),
          pl.BlockSpec((_NC, _BR, _LANES), lambda i: (0, i, 0)),
      ],
      out_specs=pl.BlockSpec((_BR, d_hid), lambda i: (i, 0)),
      out_shape=jax.ShapeDtypeStruct((npad, d_hid), jnp.float32),
  )(xp, W1, degp)

  acc1 = _make_agg_kernel(e, d_hid)(src, dst3d, hs1)

  hs2 = pl.pallas_call(
      _tc_d_body,
      grid=grid,
      in_specs=[
          pl.BlockSpec((_NC, _BR, d_hid), lambda i: (0, i, 0)),
          pl.BlockSpec((_BR, d_hid), lambda i: (i, 0)),
          pl.BlockSpec((_NC, _BR, _LANES), lambda i: (0, i, 0)),
          pl.BlockSpec((1, d_hid), lambda i: (0, 0)),
          pl.BlockSpec((d_hid, d2), lambda i: (0, 0)),
      ],
      out_specs=pl.BlockSpec((_BR, d2), lambda i: (i, 0)),
      out_shape=jax.ShapeDtypeStruct((npad, d2), jnp.float32),
  )(acc1, hs1, degp, b1r, w2p)

  acc2 = _make_agg_kernel(e, d2)(src, dst3d, hs2)

  out64 = pl.pallas_call(
      functools.partial(_tc_f_body, d_out=d_out),
      grid=grid,
      in_specs=[
          pl.BlockSpec((_NC, _BR, d2), lambda i: (0, i, 0)),
          pl.BlockSpec((_BR, d2), lambda i: (i, 0)),
          pl.BlockSpec((_NC, _BR, _LANES), lambda i: (0, i, 0)),
          pl.BlockSpec((1, d2), lambda i: (0, 0)),
      ],
      out_specs=pl.BlockSpec((_BR, d2), lambda i: (i, 0)),
      out_shape=jax.ShapeDtypeStruct((npad, d2), jnp.float32),
  )(acc2, hs2, degp, b2r)

  return out64[:n, :d_out]


# speedup vs baseline: 24.6094x; 1.0328x over previous
import functools

import jax
import jax.numpy as jnp
from jax import lax
from jax.experimental import pallas as pl
from jax.experimental.pallas import tpu as pltpu
from jax.experimental.pallas import tpu_sc as plsc

_NC = 2
_NS = 16
_W = _NC * _NS
_K = 80
_LANES = 16
_NPAD = 10240


def _zero_fill(ref, nrows, ncols):
  z = jnp.zeros((_LANES,), jnp.float32)

  def body(r, carry):
    for c in range(ncols // _LANES):
      ref[r, pl.ds(c * _LANES, _LANES)] = z
    return carry

  lax.fori_loop(0, nrows, body, 0)


def _ones_fill(ref, nrows, ncols):
  o = jnp.ones((_LANES,), jnp.float32)

  def body(r, carry):
    for c in range(ncols // _LANES):
      ref[r, pl.ds(c * _LANES, _LANES)] = o
    return carry

  lax.fori_loop(0, nrows, body, 0)


def _make_deg_kernel(e):
  n = _NPAD
  epw = e // _W
  c_chunks = epw // _K
  rpt = n // _NS
  zrows = 16
  mesh = plsc.VectorSubcoreMesh(core_axis_name="c", subcore_axis_name="s")

  def body(dst_hbm, out_hbm, acc_sp, zb, onesb, dstv, sem_s):
    cid = lax.axis_index("c")
    sid = lax.axis_index("s")
    w = cid * _NS + sid
    base = sid * rpt
    _zero_fill(zb, zrows, _LANES)

    def zcopy(i, carry):
      off = pl.multiple_of(base + i * zrows, zrows)
      pltpu.sync_copy(zb, acc_sp.at[pl.ds(off, zrows), :])
      return carry

    lax.fori_loop(0, rpt // zrows, zcopy, 0)
    _ones_fill(onesb, _K, _LANES)
    pltpu.sync_copy(dst_hbm.at[w], dstv)
    plsc.subcore_barrier()

    def chunk5(t, carry):
      descs = []
      for u in range(5):
        dsc = pltpu.make_async_copy(onesb, acc_sp.at[dstv.at[5 * t + u]],
                                    sem_s)
        dsc.start(add=True)
        descs.append(dsc)
      for dsc in descs:
        dsc.wait()
      return carry

    lax.fori_loop(0, c_chunks // 5, chunk5, 0)
    plsc.subcore_barrier()
    pltpu.sync_copy(acc_sp.at[pl.ds(base, rpt), :],
                    out_hbm.at[cid, pl.ds(base, rpt), :])

  return pl.kernel(
      body,
      out_type=jax.ShapeDtypeStruct((_NC, n, _LANES), jnp.float32),
      mesh=mesh,
      scratch_types=[
          pltpu.VMEM_SHARED((n, _LANES), jnp.float32),
          pltpu.VMEM((zrows, _LANES), jnp.float32),
          pltpu.VMEM((_K, _LANES), jnp.float32),
          pltpu.VMEM((c_chunks, _K), jnp.int32),
          pltpu.SemaphoreType.DMA,
      ],
  )


def _make_agg_kernel(e, d):
  n = _NPAD
  epw = e // _W
  c_chunks = epw // _K
  assert c_chunks % 2 == 1
  rpt = n // _NS
  zrows = 16
  mesh = plsc.VectorSubcoreMesh(core_axis_name="c", subcore_axis_name="s")

  def body(src_hbm, dst_hbm, h_hbm, out_hbm, acc_sp, zb, dstv, srcv, rows_a,
           rows_b, sem_a, sem_b, sem_sa, sem_sb):
    cid = lax.axis_index("c")
    sid = lax.axis_index("s")
    w = cid * _NS + sid
    base = sid * rpt
    _zero_fill(zb, zrows, d)

    def zcopy(i, carry):
      off = pl.multiple_of(base + i * zrows, zrows)
      pltpu.sync_copy(zb, acc_sp.at[pl.ds(off, zrows), :])
      return carry

    lax.fori_loop(0, rpt // zrows, zcopy, 0)
    pltpu.sync_copy(src_hbm.at[pl.ds(w * epw, epw)], srcv)
    pltpu.sync_copy(dst_hbm.at[w], dstv)
    plsc.subcore_barrier()

    def gather(j, rows, sem):
      off = pl.multiple_of(j * _K, _K)
      return pltpu.make_async_copy(
          h_hbm.at[srcv.at[pl.ds(off, _K)]], rows, sem)

    def scatter(j, rows, sem):
      return pltpu.make_async_copy(rows, acc_sp.at[dstv.at[j]], sem)

    def pair(t, carry):
      ja = 2 * t
      jb = 2 * t + 1
      da = gather(ja, rows_a, sem_a)
      db = gather(jb, rows_b, sem_b)
      da.start()
      db.start()
      da.wait()
      sa = scatter(ja, rows_a, sem_sa)
      sa.start(add=True)
      db.wait()
      sb = scatter(jb, rows_b, sem_sb)
      sb.start(add=True)
      sa.wait()
      sb.wait()
      return carry

    lax.fori_loop(0, (c_chunks - 1) // 2, pair, 0)
    dl = gather(c_chunks - 1, rows_a, sem_a)
    dl.start()
    dl.wait()
    sl = scatter(c_chunks - 1, rows_a, sem_sa)
    sl.start(add=True)
    sl.wait()
    plsc.subcore_barrier()
    pltpu.sync_copy(acc_sp.at[pl.ds(base, rpt), :],
                    out_hbm.at[cid, pl.ds(base, rpt), :])

  return pl.kernel(
      body,
      out_type=jax.ShapeDtypeStruct((_NC, n, d), jnp.float32),
      mesh=mesh,
      scratch_types=[
          pltpu.VMEM_SHARED((n, d), jnp.float32),
          pltpu.VMEM((zrows, d), jnp.float32),
          pltpu.VMEM((c_chunks, _K), jnp.int32),
          pltpu.VMEM((epw,), jnp.int32),
          pltpu.VMEM((_K, d), jnp.float32),
          pltpu.VMEM((_K, d), jnp.float32),
          pltpu.SemaphoreType.DMA,
          pltpu.SemaphoreType.DMA,
          pltpu.SemaphoreType.DMA,
          pltpu.SemaphoreType.DMA,
      ],
  )


_BR = 1024


def _dinv_from(degp):
  deg = jnp.sum(degp, axis=(0, 2)) + 1.0
  return lax.rsqrt(deg)


def _tc_b_body(x_ref, w_ref, degp_ref, out_ref):
  dinv = _dinv_from(degp_ref[...])
  h = jnp.dot(x_ref[...], w_ref[...], preferred_element_type=jnp.float32)
  out_ref[...] = h * dinv[:, None]


def _tc_d_body(accp_ref, hs1_ref, degp_ref, b1_ref, w2_ref, out_ref):
  dinv = _dinv_from(degp_ref[...])
  accp = accp_ref[...]
  z = dinv[:, None] * (accp[0] + accp[1] + hs1_ref[...]) + b1_ref[...]
  s = jax.nn.sigmoid(z)
  out_ref[...] = jnp.dot(
      s, w2_ref[...], preferred_element_type=jnp.float32) * dinv[:, None]


def _tc_f_body(accp_ref, hs2_ref, degp_ref, b2_ref, out_ref, *, d_out):
  dinv = _dinv_from(degp_ref[...])
  accp = accp_ref[...]
  z = dinv[:, None] * (accp[0] + accp[1] + hs2_ref[...]) + b2_ref[...]
  col = lax.broadcasted_iota(jnp.int32, z.shape, 1)
  mask = col < d_out
  zm = jnp.where(mask, z, jnp.float32(-1e30))
  m = jnp.max(zm, axis=1, keepdims=True)
  ex = jnp.where(mask, jnp.exp(zm - m), 0.0)
  lse = jnp.log(jnp.sum(ex, axis=1, keepdims=True))
  out_ref[...] = zm - m - lse


def kernel(x, edge_index, W1, b1, W2, b2):
  n, d_in = x.shape
  e = edge_index.shape[1]
  d_hid = W1.shape[1]
  d_out = W2.shape[1]
  d2 = 128
  npad = _NPAD

  xp = jnp.pad(x, ((0, npad - n), (0, 0)))
  src = edge_index[0]
  dst3d = edge_index[1].reshape(_W, -1, _K)

  w2p = jnp.pad(W2, ((0, 0), (0, d2 - d_out)))
  b1r = b1.reshape(1, d_hid)
  b2r = jnp.pad(b2, (0, d2 - d_out)).reshape(1, d2)

  degp = _make_deg_kernel(e)(dst3d)

  grid = (npad // _BR,)
  hs1 = pl.pallas_call(
      _tc_b_body,
      grid=grid,
      in_specs=[
          pl.BlockSpec((_BR, d_in), lambda i: (i, 0)),
          pl.BlockSpec((d_in, d_hid), lambda i: (0, 0)),
          pl.BlockSpec((_NC, _BR, _LANES), lambda i: (0, i, 0)),
      ],
      out_specs=pl.BlockSpec((_BR, d_hid), lambda i: (i, 0)),
      out_shape=jax.ShapeDtypeStruct((npad, d_hid), jnp.float32),
  )(xp, W1, degp)

  acc1 = _make_agg_kernel(e, d_hid)(src, dst3d, hs1)

  hs2 = pl.pallas_call(
      _tc_d_body,
      grid=grid,
      in_specs=[
          pl.BlockSpec((_NC, _BR, d_hid), lambda i: (0, i, 0)),
          pl.BlockSpec((_BR, d_hid), lambda i: (i, 0)),
          pl.BlockSpec((_NC, _BR, _LANES), lambda i: (0, i, 0)),
          pl.BlockSpec((1, d_hid), lambda i: (0, 0)),
          pl.BlockSpec((d_hid, d2), lambda i: (0, 0)),
      ],
      out_specs=pl.BlockSpec((_BR, d2), lambda i: (i, 0)),
      out_shape=jax.ShapeDtypeStruct((npad, d2), jnp.float32),
  )(acc1, hs1, degp, b1r, w2p)

  acc2 = _make_agg_kernel(e, d2)(src, dst3d, hs2)

  out64 = pl.pallas_call(
      functools.partial(_tc_f_body, d_out=d_out),
      grid=grid,
      in_specs=[
          pl.BlockSpec((_NC, _BR, d2), lambda i: (0, i, 0)),
          pl.BlockSpec((_BR, d2), lambda i: (i, 0)),
          pl.BlockSpec((_NC, _BR, _LANES), lambda i: (0, i, 0)),
          pl.BlockSpec((1, d2), lambda i: (0, 0)),
      ],
      out_specs=pl.BlockSpec((_BR, d2), lambda i: (i, 0)),
      out_shape=jax.ShapeDtypeStruct((npad, d2), jnp.float32),
  )(acc2, hs2, degp, b2r)

  return out64[:n, :d_out]


# speedup vs baseline: 24.9337x; 1.0132x over previous
import functools

import jax
import jax.numpy as jnp
from jax import lax
from jax.experimental import pallas as pl
from jax.experimental.pallas import tpu as pltpu
from jax.experimental.pallas import tpu_sc as plsc

_NC = 2
_NS = 16
_W = _NC * _NS
_K = 80
_LANES = 16
_NPAD = 10240


def _zero_fill(ref, nrows, ncols):
  z = jnp.zeros((_LANES,), jnp.float32)

  def body(r, carry):
    for c in range(ncols // _LANES):
      ref[r, pl.ds(c * _LANES, _LANES)] = z
    return carry

  lax.fori_loop(0, nrows, body, 0)


def _ones_fill(ref, nrows, ncols):
  o = jnp.ones((_LANES,), jnp.float32)

  def body(r, carry):
    for c in range(ncols // _LANES):
      ref[r, pl.ds(c * _LANES, _LANES)] = o
    return carry

  lax.fori_loop(0, nrows, body, 0)


def _make_deg_kernel(e):
  n = _NPAD
  epw = e // _W
  c_chunks = epw // _K
  rpt = n // _NS
  zrows = 16
  mesh = plsc.VectorSubcoreMesh(core_axis_name="c", subcore_axis_name="s")

  def body(dst_hbm, out_hbm, acc_sp, zb, onesb, dstv, sem_s):
    cid = lax.axis_index("c")
    sid = lax.axis_index("s")
    w = cid * _NS + sid
    base = sid * rpt
    _zero_fill(zb, zrows, _LANES)

    def zcopy(i, carry):
      off = pl.multiple_of(base + i * zrows, zrows)
      pltpu.sync_copy(zb, acc_sp.at[pl.ds(off, zrows), :])
      return carry

    lax.fori_loop(0, rpt // zrows, zcopy, 0)
    _ones_fill(onesb, _K, _LANES)
    pltpu.sync_copy(dst_hbm.at[w], dstv)
    plsc.subcore_barrier()

    def chunk5(t, carry):
      descs = []
      for u in range(5):
        dsc = pltpu.make_async_copy(onesb, acc_sp.at[dstv.at[5 * t + u]],
                                    sem_s)
        dsc.start(add=True)
        descs.append(dsc)
      for dsc in descs:
        dsc.wait()
      return carry

    lax.fori_loop(0, c_chunks // 5, chunk5, 0)
    plsc.subcore_barrier()
    pltpu.sync_copy(acc_sp.at[pl.ds(base, rpt), :],
                    out_hbm.at[cid, pl.ds(base, rpt), :])

  return pl.kernel(
      body,
      out_type=jax.ShapeDtypeStruct((_NC, n, _LANES), jnp.float32),
      mesh=mesh,
      scratch_types=[
          pltpu.VMEM_SHARED((n, _LANES), jnp.float32),
          pltpu.VMEM((zrows, _LANES), jnp.float32),
          pltpu.VMEM((_K, _LANES), jnp.float32),
          pltpu.VMEM((c_chunks, _K), jnp.int32),
          pltpu.SemaphoreType.DMA,
      ],
  )


def _make_agg_kernel(e, d):
  n = _NPAD
  epw = e // _W
  c_chunks = epw // _K
  assert c_chunks % 4 == 1
  rpt = n // _NS
  zrows = 16
  mesh = plsc.VectorSubcoreMesh(core_axis_name="c", subcore_axis_name="s")

  def body(src_hbm, dst_hbm, h_hbm, out_hbm, acc_sp, zb, dstv, srcv, rows_a,
           rows_b, sem_a, sem_b, sem_sa, sem_sb):
    cid = lax.axis_index("c")
    sid = lax.axis_index("s")
    w = cid * _NS + sid
    base = sid * rpt
    _zero_fill(zb, zrows, d)

    def zgroup(g, carry):
      descs = []
      for u in range(8):
        off = pl.multiple_of(base + (8 * g + u) * zrows, zrows)
        dsc = pltpu.make_async_copy(zb, acc_sp.at[pl.ds(off, zrows), :],
                                    sem_sa)
        dsc.start()
        descs.append(dsc)
      for dsc in descs:
        dsc.wait()
      return carry

    lax.fori_loop(0, rpt // zrows // 8, zgroup, 0)
    pltpu.sync_copy(src_hbm.at[pl.ds(w * epw, epw)], srcv)
    pltpu.sync_copy(dst_hbm.at[w], dstv)
    plsc.subcore_barrier()

    def gather(j, rows, sem):
      off = pl.multiple_of(j * _K, _K)
      return pltpu.make_async_copy(
          h_hbm.at[srcv.at[pl.ds(off, _K)]], rows, sem)

    def scatter(j, rows, sem):
      return pltpu.make_async_copy(rows, acc_sp.at[dstv.at[j]], sem)

    def quad(t, carry):
      j0 = 4 * t
      da = gather(j0, rows_a, sem_a)
      db = gather(j0 + 1, rows_b, sem_b)
      da.start()
      db.start()
      da.wait()
      sa = scatter(j0, rows_a, sem_sa)
      sa.start(add=True)
      db.wait()
      sb = scatter(j0 + 1, rows_b, sem_sb)
      sb.start(add=True)
      sa.wait()
      dc = gather(j0 + 2, rows_a, sem_a)
      dc.start()
      sb.wait()
      dd = gather(j0 + 3, rows_b, sem_b)
      dd.start()
      dc.wait()
      sc = scatter(j0 + 2, rows_a, sem_sa)
      sc.start(add=True)
      dd.wait()
      sd = scatter(j0 + 3, rows_b, sem_sb)
      sd.start(add=True)
      sc.wait()
      sd.wait()
      return carry

    lax.fori_loop(0, c_chunks // 4, quad, 0)
    dl = gather(c_chunks - 1, rows_a, sem_a)
    dl.start()
    dl.wait()
    sl = scatter(c_chunks - 1, rows_a, sem_sa)
    sl.start(add=True)
    sl.wait()
    plsc.subcore_barrier()
    pltpu.sync_copy(acc_sp.at[pl.ds(base, rpt), :],
                    out_hbm.at[cid, pl.ds(base, rpt), :])

  return pl.kernel(
      body,
      out_type=jax.ShapeDtypeStruct((_NC, n, d), jnp.float32),
      mesh=mesh,
      scratch_types=[
          pltpu.VMEM_SHARED((n, d), jnp.float32),
          pltpu.VMEM((zrows, d), jnp.float32),
          pltpu.VMEM((c_chunks, _K), jnp.int32),
          pltpu.VMEM((epw,), jnp.int32),
          pltpu.VMEM((_K, d), jnp.float32),
          pltpu.VMEM((_K, d), jnp.float32),
          pltpu.SemaphoreType.DMA,
          pltpu.SemaphoreType.DMA,
          pltpu.SemaphoreType.DMA,
          pltpu.SemaphoreType.DMA,
      ],
  )


_BR = 1024


def _dinv_from(degp):
  deg = jnp.sum(degp, axis=(0, 2)) + 1.0
  return lax.rsqrt(deg)


def _tc_b_body(x_ref, w_ref, degp_ref, out_ref):
  dinv = _dinv_from(degp_ref[...])
  h = jnp.dot(x_ref[...], w_ref[...], preferred_element_type=jnp.float32)
  out_ref[...] = h * dinv[:, None]


def _tc_d_body(accp_ref, hs1_ref, degp_ref, b1_ref, w2_ref, out_ref):
  dinv = _dinv_from(degp_ref[...])
  accp = accp_ref[...]
  z = dinv[:, None] * (accp[0] + accp[1] + hs1_ref[...]) + b1_ref[...]
  s = jax.nn.sigmoid(z)
  out_ref[...] = jnp.dot(
      s, w2_ref[...], preferred_element_type=jnp.float32) * dinv[:, None]


def _tc_f_body(accp_ref, hs2_ref, degp_ref, b2_ref, out_ref, *, d_out):
  dinv = _dinv_from(degp_ref[...])
  accp = accp_ref[...]
  z = dinv[:, None] * (accp[0] + accp[1] + hs2_ref[...]) + b2_ref[...]
  col = lax.broadcasted_iota(jnp.int32, z.shape, 1)
  mask = col < d_out
  zm = jnp.where(mask, z, jnp.float32(-1e30))
  m = jnp.max(zm, axis=1, keepdims=True)
  ex = jnp.where(mask, jnp.exp(zm - m), 0.0)
  lse = jnp.log(jnp.sum(ex, axis=1, keepdims=True))
  out_ref[...] = zm - m - lse


def kernel(x, edge_index, W1, b1, W2, b2):
  n, d_in = x.shape
  e = edge_index.shape[1]
  d_hid = W1.shape[1]
  d_out = W2.shape[1]
  d2 = 128
  npad = _NPAD

  xp = jnp.pad(x, ((0, npad - n), (0, 0)))
  src = edge_index[0]
  dst3d = edge_index[1].reshape(_W, -1, _K)

  w2p = jnp.pad(W2, ((0, 0), (0, d2 - d_out)))
  b1r = b1.reshape(1, d_hid)
  b2r = jnp.pad(b2, (0, d2 - d_out)).reshape(1, d2)

  degp = _make_deg_kernel(e)(dst3d)

  grid = (npad // _BR,)
  hs1 = pl.pallas_call(
      _tc_b_body,
      grid=grid,
      in_specs=[
          pl.BlockSpec((_BR, d_in), lambda i: (i, 0)),
          pl.BlockSpec((d_in, d_hid), lambda i: (0, 0)),
          pl.BlockSpec((_NC, _BR, _LANES), lambda i: (0, i, 0)),
      ],
      out_specs=pl.BlockSpec((_BR, d_hid), lambda i: (i, 0)),
      out_shape=jax.ShapeDtypeStruct((npad, d_hid), jnp.float32),
  )(xp, W1, degp)

  acc1 = _make_agg_kernel(e, d_hid)(src, dst3d, hs1)

  hs2 = pl.pallas_call(
      _tc_d_body,
      grid=grid,
      in_specs=[
          pl.BlockSpec((_NC, _BR, d_hid), lambda i: (0, i, 0)),
          pl.BlockSpec((_BR, d_hid), lambda i: (i, 0)),
          pl.BlockSpec((_NC, _BR, _LANES), lambda i: (0, i, 0)),
          pl.BlockSpec((1, d_hid), lambda i: (0, 0)),
          pl.BlockSpec((d_hid, d2), lambda i: (0, 0)),
      ],
      out_specs=pl.BlockSpec((_BR, d2), lambda i: (i, 0)),
      out_shape=jax.ShapeDtypeStruct((npad, d2), jnp.float32),
  )(acc1, hs1, degp, b1r, w2p)

  acc2 = _make_agg_kernel(e, d2)(src, dst3d, hs2)

  out64 = pl.pallas_call(
      functools.partial(_tc_f_body, d_out=d_out),
      grid=grid,
      in_specs=[
          pl.BlockSpec((_NC, _BR, d2), lambda i: (0, i, 0)),
          pl.BlockSpec((_BR, d2), lambda i: (i, 0)),
          pl.BlockSpec((_NC, _BR, _LANES), lambda i: (0, i, 0)),
          pl.BlockSpec((1, d2), lambda i: (0, 0)),
      ],
      out_specs=pl.BlockSpec((_BR, d2), lambda i: (i, 0)),
      out_shape=jax.ShapeDtypeStruct((npad, d2), jnp.float32),
  )(acc2, hs2, degp, b2r)

  return out64[:n, :d_out]
